# baseline (speedup 1.0000x reference)
import functools

import jax
import jax.numpy as jnp
from jax import lax
from jax.experimental import pallas as pl
from jax.experimental.pallas import tpu as pltpu
from jax.experimental.pallas import tpu_sc as plsc

PS = 64
K = 64
B = 8
G = 16
NP = G * G
ROWS_PER_PATCH = PS
ROW_W = PS * 3
N_TABLE_ROWS = B * 1024 * G
N_OUT_ROWS = B * K * ROWS_PER_PATCH


def _pool_body(hm_ref, out_ref):
    g = pl.program_id(1)
    x = hm_ref[0]
    colsum = jnp.sum(x, axis=0, keepdims=True)
    r = lax.broadcasted_iota(jnp.int32, (1024, G), 0)
    c = lax.broadcasted_iota(jnp.int32, (1024, G), 1)
    grp = ((r // PS) == c).astype(jnp.float32)
    row = lax.dot_general(colsum, grp, (((1,), (0,)), ((), ())),
                          precision=lax.Precision.HIGHEST,
                          preferred_element_type=jnp.float32)
    out_ref[0, pl.ds(g, 1), :] = row * (1.0 / (PS * PS))


def _pool(hm):
    return pl.pallas_call(
        _pool_body,
        grid=(B, G),
        in_specs=[pl.BlockSpec((1, PS, 1024), lambda b, g: (b, g, 0))],
        out_specs=pl.BlockSpec((1, G, G), lambda b, g: (b, 0, 0)),
        out_shape=jax.ShapeDtypeStruct((B, G, G), jnp.float32),
    )(hm)


def _col_of(row_vec, n):
    i = lax.broadcasted_iota(jnp.int32, (n, n), 0)
    j = lax.broadcasted_iota(jnp.int32, (n, n), 1)
    diag = (i == j).astype(row_vec.dtype)
    return jnp.sum(diag * row_vec, axis=1, keepdims=True)


def _topk_body(avg_ref, out_ref):
    b = pl.program_id(0)
    val = avg_ref[0]
    val_col = _col_of(val, NP)
    i_lane = lax.broadcasted_iota(jnp.int32, (NP, NP), 1)
    j_sub = lax.broadcasted_iota(jnp.int32, (NP, NP), 0)
    beats = (val_col > val) | ((val_col == val) & (j_sub < i_lane))
    rank = jnp.sum(beats.astype(jnp.float32), axis=0, keepdims=True)
    rank_col = _col_of(rank, NP)
    p = lax.broadcasted_iota(jnp.int32, (1, K), 1).astype(jnp.float32)
    onehot = (rank_col == p).astype(jnp.float32)
    i_col = lax.broadcasted_iota(jnp.int32, (NP, 1), 0).astype(jnp.float32)
    g_row = jnp.sum(onehot * i_col, axis=0, keepdims=True)
    g_col = _col_of(g_row, K).astype(jnp.int32)
    gh = g_col >> 4
    gw = g_col & 15
    base = b * (1024 * G) + gh * (PS * G) + gw
    step = lax.broadcasted_iota(jnp.int32, (1, ROWS_PER_PATCH), 1) * G
    out_ref[:] = base + step


def _topk_rows(avg):
    return pl.pallas_call(
        _topk_body,
        grid=(B,),
        in_specs=[pl.BlockSpec((1, 1, NP), lambda b: (b, 0, 0))],
        out_specs=pl.BlockSpec((K, ROWS_PER_PATCH), lambda b: (b, 0)),
        out_shape=jax.ShapeDtypeStruct((B * K, ROWS_PER_PATCH), jnp.int32),
    )(avg)


CHUNK = 128


def _gather_body(num_cores, rows_per_worker, table_hbm, idx_hbm, out_hbm,
                 idx_v, buf0, buf1, sem0, sem1):
    n_chunks = rows_per_worker // CHUNK
    wid = lax.axis_index("s") * num_cores + lax.axis_index("c")
    pltpu.sync_copy(idx_hbm.at[wid], idx_v)
    bufs = (buf0, buf1)
    sems = (sem0, sem1)
    cps = [None, None]
    cps[0] = pltpu.async_copy(table_hbm.at[idx_v.at[0]], bufs[0], sems[0])
    for c in range(n_chunks):
        nxt = (c + 1) % 2
        if c + 1 < n_chunks:
            cps[nxt] = pltpu.async_copy(
                table_hbm.at[idx_v.at[c + 1]], bufs[nxt], sems[nxt])
        cps[c % 2].wait()
        pltpu.sync_copy(
            bufs[c % 2],
            out_hbm.at[pl.ds(wid * rows_per_worker + c * CHUNK, CHUNK)])


def _gather(table, idx_rows):
    info = plsc.get_sparse_core_info()
    nw = info.num_cores * info.num_subcores
    rows_per_worker = N_OUT_ROWS // nw
    idx3 = idx_rows.reshape(nw, rows_per_worker // CHUNK, CHUNK)
    mesh = plsc.VectorSubcoreMesh(core_axis_name="c", subcore_axis_name="s")
    body = functools.partial(_gather_body, info.num_cores, rows_per_worker)
    return pl.kernel(
        body,
        out_type=jax.ShapeDtypeStruct((N_OUT_ROWS, ROW_W), jnp.float32),
        mesh=mesh,
        compiler_params=pltpu.CompilerParams(use_tc_tiling_on_sc=False),
        scratch_types=[
            pltpu.VMEM((rows_per_worker // CHUNK, CHUNK), jnp.int32),
            pltpu.VMEM((CHUNK, ROW_W), jnp.float32),
            pltpu.VMEM((CHUNK, ROW_W), jnp.float32),
            pltpu.SemaphoreType.DMA,
            pltpu.SemaphoreType.DMA,
        ],
    )(table, idx3)


def kernel(heatmap, image):
    hm = heatmap.reshape(B, 1024, 1024)
    avg = _pool(hm).reshape(B, 1, NP)
    rows = _topk_rows(avg)
    table = image.reshape(N_TABLE_ROWS, ROW_W)
    out = _gather(table, rows.reshape(-1))
    return out.reshape(B * K, PS, PS, 3)


# speedup vs baseline: 5.6911x; 5.6911x over previous
import functools

import jax
import jax.numpy as jnp
from jax import lax
from jax.experimental import pallas as pl
from jax.experimental.pallas import tpu as pltpu
from jax.experimental.pallas import tpu_sc as plsc

PS = 64
K = 64
B = 8
G = 16
NP = G * G
ROWS_PER_PATCH = PS
ROW_W = PS * 3
N_TABLE_ROWS = B * 1024 * G
N_OUT_ROWS = B * K * ROWS_PER_PATCH


def _pool_body(hm_ref, out_ref):
    g = pl.program_id(1)
    x = hm_ref[0]
    colsum = jnp.sum(x, axis=0, keepdims=True)
    r = lax.broadcasted_iota(jnp.int32, (1024, G), 0)
    c = lax.broadcasted_iota(jnp.int32, (1024, G), 1)
    grp = ((r // PS) == c).astype(jnp.float32)
    row = lax.dot_general(colsum, grp, (((1,), (0,)), ((), ())),
                          precision=lax.Precision.HIGHEST,
                          preferred_element_type=jnp.float32)
    out_ref[0, pl.ds(g, 1), :] = row * (1.0 / (PS * PS))


def _pool(hm):
    return pl.pallas_call(
        _pool_body,
        grid=(B, G),
        in_specs=[pl.BlockSpec((1, PS, 1024), lambda b, g: (b, g, 0))],
        out_specs=pl.BlockSpec((1, G, G), lambda b, g: (b, 0, 0)),
        out_shape=jax.ShapeDtypeStruct((B, G, G), jnp.float32),
    )(hm)


def _col_of(row_vec, n):
    i = lax.broadcasted_iota(jnp.int32, (n, n), 0)
    j = lax.broadcasted_iota(jnp.int32, (n, n), 1)
    diag = (i == j).astype(row_vec.dtype)
    return jnp.sum(diag * row_vec, axis=1, keepdims=True)


def _topk_body(avg_ref, out_ref):
    b = pl.program_id(0)
    val = avg_ref[0]
    val_col = _col_of(val, NP)
    i_lane = lax.broadcasted_iota(jnp.int32, (NP, NP), 1)
    j_sub = lax.broadcasted_iota(jnp.int32, (NP, NP), 0)
    beats = (val_col > val) | ((val_col == val) & (j_sub < i_lane))
    rank = jnp.sum(beats.astype(jnp.float32), axis=0, keepdims=True)
    rank_col = _col_of(rank, NP)
    p = lax.broadcasted_iota(jnp.int32, (1, K), 1).astype(jnp.float32)
    onehot = (rank_col == p).astype(jnp.float32)
    i_col = lax.broadcasted_iota(jnp.int32, (NP, 1), 0).astype(jnp.float32)
    g_row = jnp.sum(onehot * i_col, axis=0, keepdims=True)
    g_col = _col_of(g_row, K).astype(jnp.int32)
    gh = g_col >> 4
    gw = g_col & 15
    base = b * (1024 * G) + gh * (PS * G) + gw
    step = lax.broadcasted_iota(jnp.int32, (1, ROWS_PER_PATCH), 1) * G
    out_ref[:] = base + step


def _topk_rows(avg):
    return pl.pallas_call(
        _topk_body,
        grid=(B,),
        in_specs=[pl.BlockSpec((1, 1, NP), lambda b: (b, 0, 0))],
        out_specs=pl.BlockSpec((K, ROWS_PER_PATCH), lambda b: (b, 0)),
        out_shape=jax.ShapeDtypeStruct((B * K, ROWS_PER_PATCH), jnp.int32),
    )(avg)


CHUNK = 128


def _gather_body(num_cores, rows_per_worker, table_hbm, idx_hbm, out_hbm,
                 idx_v, buf0, buf1, sem0, sem1):
    n_chunks = rows_per_worker // CHUNK
    wid = lax.axis_index("s") * num_cores + lax.axis_index("c")
    pltpu.sync_copy(idx_hbm.at[wid], idx_v)
    bufs = (buf0, buf1)
    sems = (sem0, sem1)
    cps = [None, None]
    cps[0] = pltpu.async_copy(table_hbm.at[idx_v.at[0]], bufs[0], sems[0])
    for c in range(n_chunks):
        nxt = (c + 1) % 2
        if c + 1 < n_chunks:
            cps[nxt] = pltpu.async_copy(
                table_hbm.at[idx_v.at[c + 1]], bufs[nxt], sems[nxt])
        cps[c % 2].wait()
        pltpu.sync_copy(
            bufs[c % 2],
            out_hbm.at[pl.ds(wid * rows_per_worker + c * CHUNK, CHUNK)])


def _gather(table, idx_rows):
    info = plsc.get_sparse_core_info()
    nw = info.num_cores * info.num_subcores
    rows_per_worker = N_OUT_ROWS // nw
    idx3 = idx_rows.reshape(nw, rows_per_worker // CHUNK, CHUNK)
    mesh = plsc.VectorSubcoreMesh(core_axis_name="c", subcore_axis_name="s")
    body = functools.partial(_gather_body, info.num_cores, rows_per_worker)
    return pl.kernel(
        body,
        out_type=jax.ShapeDtypeStruct((N_OUT_ROWS, ROW_W), jnp.float32),
        mesh=mesh,
        compiler_params=pltpu.CompilerParams(use_tc_tiling_on_sc=False),
        scratch_types=[
            pltpu.VMEM((rows_per_worker // CHUNK, CHUNK), jnp.int32),
            pltpu.VMEM((CHUNK, ROW_W), jnp.float32),
            pltpu.VMEM((CHUNK, ROW_W), jnp.float32),
            pltpu.SemaphoreType.DMA,
            pltpu.SemaphoreType.DMA,
        ],
    )(table, idx3)


def _tc_gather_body(idx_ref, img_ref, out_ref):
    out_ref[...] = img_ref[...]


def _tc_gather(image, idx):
    def img_map(n, idx_ref):
        g = idx_ref[n]
        return (n // K, g >> 4, g & 15, 0)

    grid_spec = pltpu.PrefetchScalarGridSpec(
        num_scalar_prefetch=1,
        grid=(B * K,),
        in_specs=[pl.BlockSpec((1, PS, PS, 3), img_map)],
        out_specs=pl.BlockSpec((1, PS, PS, 3), lambda n, idx_ref: (n, 0, 0, 0)),
    )
    return pl.pallas_call(
        _tc_gather_body,
        grid_spec=grid_spec,
        out_shape=jax.ShapeDtypeStruct((B * K, PS, PS, 3), jnp.float32),
    )(idx, image)


def kernel(heatmap, image):
    hm = heatmap.reshape(B, 1024, 1024)
    avg = _pool(hm).reshape(B, 1, NP)
    rows = _topk_rows(avg)
    out = _tc_gather(image, _rows_to_g(rows))
    return out


def _rows_to_g(rows):
    row0 = rows[:, 0] % (1024 * G)
    return ((row0 >> 10) << 4) | (row0 & 15)


# speedup vs baseline: 20.1854x; 3.5468x over previous
import functools

import jax
import jax.numpy as jnp
from jax import lax
from jax.experimental import pallas as pl
from jax.experimental.pallas import tpu as pltpu
from jax.experimental.pallas import tpu_sc as plsc

PS = 64
K = 64
B = 8
G = 16
NP = G * G
ROWS_PER_PATCH = PS
ROW_W = PS * 3
N_TABLE_ROWS = B * 1024 * G
N_OUT_ROWS = B * K * ROWS_PER_PATCH


def _pool_body(hm_ref, out_ref):
    g = pl.program_id(1)
    x = hm_ref[0]
    colsum = jnp.sum(x, axis=0, keepdims=True)
    r = lax.broadcasted_iota(jnp.int32, (1024, G), 0)
    c = lax.broadcasted_iota(jnp.int32, (1024, G), 1)
    grp = ((r // PS) == c).astype(jnp.float32)
    row = lax.dot_general(colsum, grp, (((1,), (0,)), ((), ())),
                          precision=lax.Precision.HIGHEST,
                          preferred_element_type=jnp.float32)
    out_ref[0, pl.ds(g, 1), :] = row * (1.0 / (PS * PS))


def _pool(hm):
    return pl.pallas_call(
        _pool_body,
        grid=(B, G),
        in_specs=[pl.BlockSpec((1, PS, 1024), lambda b, g: (b, g, 0))],
        out_specs=pl.BlockSpec((1, G, G), lambda b, g: (b, 0, 0)),
        out_shape=jax.ShapeDtypeStruct((B, G, G), jnp.float32),
    )(hm)


def _col_of(row_vec, n):
    i = lax.broadcasted_iota(jnp.int32, (n, n), 0)
    j = lax.broadcasted_iota(jnp.int32, (n, n), 1)
    diag = (i == j).astype(row_vec.dtype)
    return jnp.sum(diag * row_vec, axis=1, keepdims=True)


def _topk_body(avg_ref, out_ref):
    b = pl.program_id(0)
    val = avg_ref[0]
    val_col = _col_of(val, NP)
    i_lane = lax.broadcasted_iota(jnp.int32, (NP, NP), 1)
    j_sub = lax.broadcasted_iota(jnp.int32, (NP, NP), 0)
    beats = (val_col > val) | ((val_col == val) & (j_sub < i_lane))
    rank = jnp.sum(beats.astype(jnp.float32), axis=0, keepdims=True)
    rank_col = _col_of(rank, NP)
    p = lax.broadcasted_iota(jnp.int32, (1, K), 1).astype(jnp.float32)
    onehot = (rank_col == p).astype(jnp.float32)
    i_col = lax.broadcasted_iota(jnp.int32, (NP, 1), 0).astype(jnp.float32)
    g_row = jnp.sum(onehot * i_col, axis=0, keepdims=True)
    g_col = _col_of(g_row, K).astype(jnp.int32)
    gh = g_col >> 4
    gw = g_col & 15
    base = b * (1024 * G) + gh * (PS * G) + gw
    step = lax.broadcasted_iota(jnp.int32, (1, ROWS_PER_PATCH), 1) * G
    out_ref[:] = base + step


def _topk_rows(avg):
    return pl.pallas_call(
        _topk_body,
        grid=(B,),
        in_specs=[pl.BlockSpec((1, 1, NP), lambda b: (b, 0, 0))],
        out_specs=pl.BlockSpec((K, ROWS_PER_PATCH), lambda b: (b, 0)),
        out_shape=jax.ShapeDtypeStruct((B * K, ROWS_PER_PATCH), jnp.int32),
    )(avg)


CHUNK = 128


def _gather_body(num_cores, rows_per_worker, table_hbm, idx_hbm, out_hbm,
                 idx_v, buf0, buf1, sem0, sem1):
    n_chunks = rows_per_worker // CHUNK
    wid = lax.axis_index("s") * num_cores + lax.axis_index("c")
    pltpu.sync_copy(idx_hbm.at[wid], idx_v)
    bufs = (buf0, buf1)
    sems = (sem0, sem1)
    cps = [None, None]
    cps[0] = pltpu.async_copy(table_hbm.at[idx_v.at[0]], bufs[0], sems[0])
    for c in range(n_chunks):
        nxt = (c + 1) % 2
        if c + 1 < n_chunks:
            cps[nxt] = pltpu.async_copy(
                table_hbm.at[idx_v.at[c + 1]], bufs[nxt], sems[nxt])
        cps[c % 2].wait()
        pltpu.sync_copy(
            bufs[c % 2],
            out_hbm.at[pl.ds(wid * rows_per_worker + c * CHUNK, CHUNK)])


def _gather(table, idx_rows):
    info = plsc.get_sparse_core_info()
    nw = info.num_cores * info.num_subcores
    rows_per_worker = N_OUT_ROWS // nw
    idx3 = idx_rows.reshape(nw, rows_per_worker // CHUNK, CHUNK)
    mesh = plsc.VectorSubcoreMesh(core_axis_name="c", subcore_axis_name="s")
    body = functools.partial(_gather_body, info.num_cores, rows_per_worker)
    return pl.kernel(
        body,
        out_type=jax.ShapeDtypeStruct((N_OUT_ROWS, ROW_W), jnp.float32),
        mesh=mesh,
        compiler_params=pltpu.CompilerParams(use_tc_tiling_on_sc=False),
        scratch_types=[
            pltpu.VMEM((rows_per_worker // CHUNK, CHUNK), jnp.int32),
            pltpu.VMEM((CHUNK, ROW_W), jnp.float32),
            pltpu.VMEM((CHUNK, ROW_W), jnp.float32),
            pltpu.SemaphoreType.DMA,
            pltpu.SemaphoreType.DMA,
        ],
    )(table, idx3)


def _tc_gather_body(idx_ref, img_ref, out_ref):
    n = pl.program_id(0)
    parity = idx_ref[n] & 1
    x = img_ref[0]
    sel = jnp.where(parity == 0, x[:, :PS], x[:, PS:])
    out_ref[0, 0] = sel


def _tc_gather(planar, idx):
    def img_map(n, c, idx_ref):
        g = idx_ref[n]
        return ((n // K) * 3 + c, g >> 4, (g & 15) >> 1)

    grid_spec = pltpu.PrefetchScalarGridSpec(
        num_scalar_prefetch=1,
        grid=(B * K, 3),
        in_specs=[pl.BlockSpec((1, PS, 2 * PS), img_map)],
        out_specs=pl.BlockSpec((1, 1, PS, PS),
                               lambda n, c, idx_ref: (c, n, 0, 0)),
    )
    return pl.pallas_call(
        _tc_gather_body,
        grid_spec=grid_spec,
        out_shape=jax.ShapeDtypeStruct((3, B * K, PS, PS), jnp.float32),
    )(idx, planar)


def kernel(heatmap, image):
    hm = heatmap.reshape(B, 1024, 1024)
    avg = _pool(hm).reshape(B, 1, NP)
    rows = _topk_rows(avg)
    planar = image.transpose(0, 3, 1, 2).reshape(B * 3, 1024, 1024)
    z = _tc_gather(planar, _rows_to_g(rows))
    return z.transpose(1, 2, 3, 0)


def _rows_to_g(rows):
    row0 = rows[:, 0] % (1024 * G)
    return ((row0 >> 10) << 4) | (row0 & 15)


# speedup vs baseline: 42.3162x; 2.0964x over previous
import functools

import jax
import jax.numpy as jnp
from jax import lax
from jax.experimental import pallas as pl
from jax.experimental.pallas import tpu as pltpu
from jax.experimental.pallas import tpu_sc as plsc

PS = 64
K = 64
B = 8
G = 16
NP = G * G
ROWS_PER_PATCH = PS
ROW_W = PS * 3
N_TABLE_ROWS = B * 1024 * G
N_OUT_ROWS = B * K * ROWS_PER_PATCH


def _pool_body(hm_ref, out_ref):
    g = pl.program_id(1)
    x = hm_ref[0]
    colsum = jnp.sum(x, axis=0, keepdims=True)
    r = lax.broadcasted_iota(jnp.int32, (1024, G), 0)
    c = lax.broadcasted_iota(jnp.int32, (1024, G), 1)
    grp = ((r // PS) == c).astype(jnp.float32)
    row = lax.dot_general(colsum, grp, (((1,), (0,)), ((), ())),
                          precision=lax.Precision.HIGHEST,
                          preferred_element_type=jnp.float32)
    out_ref[0, pl.ds(g, 1), :] = row * (1.0 / (PS * PS))


def _pool(hm):
    return pl.pallas_call(
        _pool_body,
        grid=(B, G),
        in_specs=[pl.BlockSpec((1, PS, 1024), lambda b, g: (b, g, 0))],
        out_specs=pl.BlockSpec((1, G, G), lambda b, g: (b, 0, 0)),
        out_shape=jax.ShapeDtypeStruct((B, G, G), jnp.float32),
    )(hm)


def _col_of(row_vec, n):
    i = lax.broadcasted_iota(jnp.int32, (n, n), 0)
    j = lax.broadcasted_iota(jnp.int32, (n, n), 1)
    diag = (i == j).astype(row_vec.dtype)
    return jnp.sum(diag * row_vec, axis=1, keepdims=True)


def _topk_body(avg_ref, out_ref):
    b = pl.program_id(0)
    val = avg_ref[0]
    val_col = _col_of(val, NP)
    i_lane = lax.broadcasted_iota(jnp.int32, (NP, NP), 1)
    j_sub = lax.broadcasted_iota(jnp.int32, (NP, NP), 0)
    beats = (val_col > val) | ((val_col == val) & (j_sub < i_lane))
    rank = jnp.sum(beats.astype(jnp.float32), axis=0, keepdims=True)
    rank_col = _col_of(rank, NP)
    p = lax.broadcasted_iota(jnp.int32, (1, K), 1).astype(jnp.float32)
    onehot = (rank_col == p).astype(jnp.float32)
    i_col = lax.broadcasted_iota(jnp.int32, (NP, 1), 0).astype(jnp.float32)
    g_row = jnp.sum(onehot * i_col, axis=0, keepdims=True)
    g_col = _col_of(g_row, K).astype(jnp.int32)
    gh = g_col >> 4
    gw = g_col & 15
    base = b * (1024 * G) + gh * (PS * G) + gw
    step = lax.broadcasted_iota(jnp.int32, (1, ROWS_PER_PATCH), 1) * G
    out_ref[:] = base + step


def _topk_rows(avg):
    return pl.pallas_call(
        _topk_body,
        grid=(B,),
        in_specs=[pl.BlockSpec((1, 1, NP), lambda b: (b, 0, 0))],
        out_specs=pl.BlockSpec((K, ROWS_PER_PATCH), lambda b: (b, 0)),
        out_shape=jax.ShapeDtypeStruct((B * K, ROWS_PER_PATCH), jnp.int32),
    )(avg)


CHUNK = 128


def _gather_body(num_cores, rows_per_worker, table_hbm, idx_hbm, out_hbm,
                 idx_v, buf0, buf1, sem0, sem1):
    n_chunks = rows_per_worker // CHUNK
    wid = lax.axis_index("s") * num_cores + lax.axis_index("c")
    pltpu.sync_copy(idx_hbm.at[wid], idx_v)
    bufs = (buf0, buf1)
    sems = (sem0, sem1)
    cps = [None, None]
    cps[0] = pltpu.async_copy(table_hbm.at[idx_v.at[0]], bufs[0], sems[0])
    for c in range(n_chunks):
        nxt = (c + 1) % 2
        if c + 1 < n_chunks:
            cps[nxt] = pltpu.async_copy(
                table_hbm.at[idx_v.at[c + 1]], bufs[nxt], sems[nxt])
        cps[c % 2].wait()
        pltpu.sync_copy(
            bufs[c % 2],
            out_hbm.at[pl.ds(wid * rows_per_worker + c * CHUNK, CHUNK)])


def _gather(table, idx_rows):
    info = plsc.get_sparse_core_info()
    nw = info.num_cores * info.num_subcores
    rows_per_worker = N_OUT_ROWS // nw
    idx3 = idx_rows.reshape(nw, rows_per_worker // CHUNK, CHUNK)
    mesh = plsc.VectorSubcoreMesh(core_axis_name="c", subcore_axis_name="s")
    body = functools.partial(_gather_body, info.num_cores, rows_per_worker)
    return pl.kernel(
        body,
        out_type=jax.ShapeDtypeStruct((N_OUT_ROWS, ROW_W), jnp.float32),
        mesh=mesh,
        compiler_params=pltpu.CompilerParams(use_tc_tiling_on_sc=False),
        scratch_types=[
            pltpu.VMEM((rows_per_worker // CHUNK, CHUNK), jnp.int32),
            pltpu.VMEM((CHUNK, ROW_W), jnp.float32),
            pltpu.VMEM((CHUNK, ROW_W), jnp.float32),
            pltpu.SemaphoreType.DMA,
            pltpu.SemaphoreType.DMA,
        ],
    )(table, idx3)


def _tc_gather_body(idx_ref, img_ref, out_ref):
    n = pl.program_id(0)
    parity = idx_ref[n] & 1
    x = img_ref[...]
    sel = jnp.where(parity == 0, x[:, :, :PS], x[:, :, PS:])
    out_ref[0] = sel


def _tc_gather(planar, idx):
    def img_map(n, idx_ref):
        g = idx_ref[n]
        return (n // K, g >> 4, (g & 15) >> 1)

    grid_spec = pltpu.PrefetchScalarGridSpec(
        num_scalar_prefetch=1,
        grid=(B * K,),
        in_specs=[pl.BlockSpec((3, PS, 2 * PS), img_map)],
        out_specs=pl.BlockSpec((1, 3, PS, PS),
                               lambda n, idx_ref: (n, 0, 0, 0)),
    )
    return pl.pallas_call(
        _tc_gather_body,
        grid_spec=grid_spec,
        out_shape=jax.ShapeDtypeStruct((B * K, 3, PS, PS), jnp.float32),
    )(idx, planar)


def kernel(heatmap, image):
    hm = heatmap.reshape(B, 1024, 1024)
    avg = _pool(hm).reshape(B, 1, NP)
    rows = _topk_rows(avg)
    planar = image.transpose(0, 3, 1, 2).reshape(B * 3, 1024, 1024)
    z = _tc_gather(planar, _rows_to_g(rows))
    return z.transpose(0, 2, 3, 1)


def _rows_to_g(rows):
    row0 = rows[:, 0] % (1024 * G)
    return ((row0 >> 10) << 4) | (row0 & 15)
